# Initial kernel scaffold; baseline (speedup 1.0000x reference)
import jax
import jax.numpy as jnp
from jax.experimental import pallas as pl


def kernel(features, tois, embeddings):
    raise NotImplementedError("write your pallas kernel here")


# speedup vs baseline: 5.3252x; 5.3252x over previous
import jax
import jax.numpy as jnp
from jax import lax
from jax.experimental import pallas as pl
from jax.experimental.pallas import tpu as pltpu
from jax.experimental.pallas import tpu_sc as plsc

_POOL = 8
_NSLOT = _POOL + 2
_NC, _NS, _LANES = 2, 16, 16
_NW = _NC * _NS


def _cs_body(ft_ref, cs_ref):
    x = ft_ref[0]
    t_len, dblk = x.shape
    ri = lax.broadcasted_iota(jnp.int32, (128, 128), 0)
    ci = lax.broadcasted_iota(jnp.int32, (128, 128), 1)
    ltri = (ri >= ci).astype(jnp.float32)
    carry = jnp.zeros((1, dblk), jnp.float32)
    pieces = [jnp.zeros((1, dblk), jnp.float32)]
    for k in range(t_len // 128):
        ik = jnp.dot(ltri, x[k * 128:(k + 1) * 128],
                     precision=lax.Precision.HIGHEST)
        blk = ik + carry
        pieces.append(blk)
        carry = blk[127:128, :]
    pieces.append(jnp.zeros((7, dblk), jnp.float32))
    cs_ref[0] = jnp.concatenate(pieces, axis=0)


def _prefix_table(ft):
    b, t, d = ft.shape
    dblk = 256
    return pl.pallas_call(
        _cs_body,
        grid=(b, d // dblk),
        in_specs=[pl.BlockSpec((1, t, dblk), lambda i, j: (i, 0, j))],
        out_specs=pl.BlockSpec((1, t + 8, dblk), lambda i, j: (i, 0, j)),
        out_shape=jax.ShapeDtypeStruct((b, t + 8, d), jnp.float32),
    )(ft)


def _sc_pool(cs_flat, starts, ends, n_span, n_per_batch, cs_rows_per_batch, d):
    spw = n_span // _NW
    rpw = spw * _NSLOT
    krows = 32
    nchunk = rpw // krows

    mesh = plsc.VectorSubcoreMesh(core_axis_name="c", subcore_axis_name="s")

    def body(cs_hbm, st_hbm, en_hbm, out_hbm, st_v, en_v, hi_v, lo_v, sc_v,
             hbuf, lbuf, sem_h, sem_l):
        wid = lax.axis_index("s") * _NC + lax.axis_index("c")
        span0 = wid * spw
        row0 = wid * rpw
        csbase = (span0 // n_per_batch) * cs_rows_per_batch
        pltpu.sync_copy(st_hbm.at[pl.ds(span0, spw)], st_v)
        pltpu.sync_copy(en_hbm.at[pl.ds(span0, spw)], en_v)

        zero16 = jnp.zeros((_LANES,), jnp.int32)
        onef = jnp.ones((_LANES,), jnp.float32)
        zerof = jnp.zeros((_LANES,), jnp.float32)

        def gbody(g, carry):
            sl = g * _LANES + lax.iota(jnp.int32, _LANES)
            start = st_v[pl.ds(g * _LANES, _LANES)]
            end = en_v[pl.ds(g * _LANES, _LANES)]
            ln = end - start
            m = jnp.maximum(ln - 2, 0)
            lm = m >> 3
            lr = m & 7
            big = ln > _POOL + 2
            inv = 1.0 / jnp.maximum(lm + lr, 1).astype(jnp.float32)
            base = sl * _NSLOT
            for s in range(_NSLOT):
                if s == 0:
                    hi, lo, sc = start + 1, start, onef
                elif s == _NSLOT - 1:
                    p = ln > 1
                    hi = jnp.where(p, end, 0)
                    lo = jnp.where(p, end - 1, 0)
                    sc = jnp.where(p, onef, zerof)
                else:
                    exact = (ln >= s + 2) & (ln <= _POOL + 2)
                    hi = jnp.where(big, start + 1 + s * lm + lr,
                                   jnp.where(exact, start + s + 1, 0))
                    lo = jnp.where(big, start + 1 + (s - 1) * lm,
                                   jnp.where(exact, start + s, 0))
                    sc = jnp.where(big, inv, jnp.where(exact, onef, zerof))
                pos = base + s
                plsc.store_scatter(hi_v, [pos], hi + csbase)
                plsc.store_scatter(lo_v, [pos], lo + csbase)
                plsc.store_scatter(sc_v, [pos], sc)
            return carry

        lax.fori_loop(0, spw // _LANES, gbody, 0)

        def cbody(c, carry):
            cph = pltpu.async_copy(cs_hbm.at[hi_v.at[pl.ds(c * krows, krows)]], hbuf, sem_h)
            cpl = pltpu.async_copy(cs_hbm.at[lo_v.at[pl.ds(c * krows, krows)]], lbuf, sem_l)
            cph.wait()
            cpl.wait()

            def rbody(r, carry2):
                sv = plsc.load_gather(sc_v, [zero16 + (c * krows + r)])
                for j in range(d // _LANES):
                    h = hbuf[r, pl.ds(j * _LANES, _LANES)]
                    l = lbuf[r, pl.ds(j * _LANES, _LANES)]
                    hbuf[r, pl.ds(j * _LANES, _LANES)] = (h - l) * sv
                return carry2

            lax.fori_loop(0, krows, rbody, 0)
            pltpu.sync_copy(hbuf, out_hbm.at[pl.ds(row0 + c * krows, krows)])
            return carry

        lax.fori_loop(0, nchunk, cbody, 0)

    return pl.kernel(
        body,
        mesh=mesh,
        compiler_params=pltpu.CompilerParams(needs_layout_passes=False),
        out_type=jax.ShapeDtypeStruct((n_span * _NSLOT, d), jnp.float32),
        scratch_types=[
            pltpu.VMEM((spw,), jnp.int32),
            pltpu.VMEM((spw,), jnp.int32),
            pltpu.VMEM((nchunk * krows,), jnp.int32),
            pltpu.VMEM((nchunk * krows,), jnp.int32),
            pltpu.VMEM((nchunk * krows,), jnp.float32),
            pltpu.VMEM((krows, d), jnp.float32),
            pltpu.VMEM((krows, d), jnp.float32),
            pltpu.SemaphoreType.DMA,
            pltpu.SemaphoreType.DMA,
        ],
    )(cs_flat, starts, ends)


def kernel(features, tois, embeddings):
    b, d, t = features.shape
    n = tois.shape[1]
    ft = jnp.transpose(features, (0, 2, 1))
    cs = _prefix_table(ft)
    cs_flat = cs.reshape(b * (t + 8), d)
    tois_flat = tois.reshape(b * n, 2).astype(jnp.int32)
    starts = tois_flat[:, 0]
    ends = tois_flat[:, 1]
    rows = _sc_pool(cs_flat, starts, ends, b * n, n, t + 8, d)
    out = rows.reshape(b * n, _NSLOT * d)
    cum = jnp.cumsum(jnp.full((b,), n, dtype=jnp.int32))
    return (out, cum)


# speedup vs baseline: 5.5994x; 1.0515x over previous
import jax
import jax.numpy as jnp
from jax import lax
from jax.experimental import pallas as pl
from jax.experimental.pallas import tpu as pltpu
from jax.experimental.pallas import tpu_sc as plsc

_POOL = 8
_NSLOT = _POOL + 2
_NC, _NS, _LANES = 2, 16, 16
_NW = _NC * _NS


def _cs_body(ft_ref, cs_ref):
    x = ft_ref[0]
    t_len, dblk = x.shape
    ri = lax.broadcasted_iota(jnp.int32, (128, 128), 0)
    ci = lax.broadcasted_iota(jnp.int32, (128, 128), 1)
    ltri = (ri >= ci).astype(jnp.float32)
    carry = jnp.zeros((1, dblk), jnp.float32)
    pieces = [jnp.zeros((1, dblk), jnp.float32)]
    for k in range(t_len // 128):
        ik = jnp.dot(ltri, x[k * 128:(k + 1) * 128],
                     precision=lax.Precision.HIGHEST)
        blk = ik + carry
        pieces.append(blk)
        carry = blk[127:128, :]
    pieces.append(jnp.zeros((7, dblk), jnp.float32))
    cs_ref[0] = jnp.concatenate(pieces, axis=0)


def _prefix_table(ft):
    b, t, d = ft.shape
    dblk = 256
    return pl.pallas_call(
        _cs_body,
        grid=(b, d // dblk),
        in_specs=[pl.BlockSpec((1, t, dblk), lambda i, j: (i, 0, j))],
        out_specs=pl.BlockSpec((1, t + 8, dblk), lambda i, j: (i, 0, j)),
        out_shape=jax.ShapeDtypeStruct((b, t + 8, d), jnp.float32),
    )(ft)


def _sc_pool(cs_flat, starts, ends, n_span, n_per_batch, cs_rows_per_batch, d):
    spw = n_span // _NW
    rpw = spw * _NSLOT
    krows = 16
    nchunk = rpw // krows

    mesh = plsc.VectorSubcoreMesh(core_axis_name="c", subcore_axis_name="s")

    def body(cs_hbm, st_hbm, en_hbm, out_hbm, st_v, en_v, cb_v, sc_v,
             buf0, buf1, gs0, gs1, ws0, ws1):
        wid = lax.axis_index("s") * _NC + lax.axis_index("c")
        span0 = wid * spw
        row0 = wid * rpw
        csbase = (span0 // n_per_batch) * cs_rows_per_batch
        pltpu.sync_copy(st_hbm.at[pl.ds(span0, spw)], st_v)
        pltpu.sync_copy(en_hbm.at[pl.ds(span0, spw)], en_v)

        zero16 = jnp.zeros((_LANES,), jnp.int32)
        onef = jnp.ones((_LANES,), jnp.float32)
        zerof = jnp.zeros((_LANES,), jnp.float32)
        bufs = (buf0, buf1)
        gsems = (gs0, gs1)
        wsems = (ws0, ws1)

        def gbody(g, carry):
            sl = g * _LANES + lax.iota(jnp.int32, _LANES)
            start = st_v[pl.ds(g * _LANES, _LANES)]
            end = en_v[pl.ds(g * _LANES, _LANES)]
            ln = end - start
            m = jnp.maximum(ln - 2, 0)
            lm = m >> 3
            lr = m & 7
            big = ln > _POOL + 2
            inv = 1.0 / jnp.maximum(lm + lr, 1).astype(jnp.float32)
            base = sl * _NSLOT
            for s in range(_NSLOT):
                if s == 0:
                    hi, lo, sc = start + 1, start, onef
                elif s == _NSLOT - 1:
                    p = ln > 1
                    hi = jnp.where(p, end, 0)
                    lo = jnp.where(p, end - 1, 0)
                    sc = jnp.where(p, onef, zerof)
                else:
                    exact = (ln >= s + 2) & (ln <= _POOL + 2)
                    hi = jnp.where(big, start + 1 + s * lm + lr,
                                   jnp.where(exact, start + s + 1, 0))
                    lo = jnp.where(big, start + 1 + (s - 1) * lm,
                                   jnp.where(exact, start + s, 0))
                    sc = jnp.where(big, inv, jnp.where(exact, onef, zerof))
                pos = base + s
                ph = ((pos >> 4) << 5) + (pos & 15)
                plsc.store_scatter(cb_v, [ph], hi + csbase)
                plsc.store_scatter(cb_v, [ph + 16], lo + csbase)
                plsc.store_scatter(sc_v, [pos], sc)
            return carry

        lax.fori_loop(0, spw // _LANES, gbody, 0)

        def gather_of(c, par):
            return pltpu.make_async_copy(
                cs_hbm.at[cb_v.at[pl.ds(c * 2 * krows, 2 * krows)]],
                bufs[par], gsems[par])

        def write_of(c, par):
            return pltpu.make_async_copy(
                bufs[par].at[pl.ds(0, krows)],
                out_hbm.at[pl.ds(row0 + c * krows, krows)], wsems[par])

        gather_of(0, 0).start()

        def cbody(cc, carry):
            for par in range(2):
                c = cc * 2 + par
                nxt = c + 1
                @pl.when(nxt < nchunk)
                def _():
                    @pl.when(c >= 1)
                    def _():
                        write_of(nxt - 2, 1 - par).wait()
                    gather_of(nxt, 1 - par).start()

                gather_of(c, par).wait()

                def rbody(r, carry2):
                    sv = plsc.load_gather(sc_v, [zero16 + (c * krows + r)])
                    for j in range(d // _LANES):
                        h = bufs[par][r, pl.ds(j * _LANES, _LANES)]
                        l = bufs[par][r + krows, pl.ds(j * _LANES, _LANES)]
                        bufs[par][r, pl.ds(j * _LANES, _LANES)] = (h - l) * sv
                    return carry2

                lax.fori_loop(0, krows, rbody, 0)
                write_of(c, par).start()
            return carry

        lax.fori_loop(0, nchunk // 2, cbody, 0)
        write_of(nchunk - 2, 0).wait()
        write_of(nchunk - 1, 1).wait()

    return pl.kernel(
        body,
        mesh=mesh,
        compiler_params=pltpu.CompilerParams(needs_layout_passes=False),
        out_type=jax.ShapeDtypeStruct((n_span * _NSLOT, d), jnp.float32),
        scratch_types=[
            pltpu.VMEM((spw,), jnp.int32),
            pltpu.VMEM((spw,), jnp.int32),
            pltpu.VMEM((nchunk * 2 * krows,), jnp.int32),
            pltpu.VMEM((nchunk * krows,), jnp.float32),
            pltpu.VMEM((2 * krows, d), jnp.float32),
            pltpu.VMEM((2 * krows, d), jnp.float32),
            pltpu.SemaphoreType.DMA,
            pltpu.SemaphoreType.DMA,
            pltpu.SemaphoreType.DMA,
            pltpu.SemaphoreType.DMA,
        ],
    )(cs_flat, starts, ends)


def kernel(features, tois, embeddings):
    b, d, t = features.shape
    n = tois.shape[1]
    ft = jnp.transpose(features, (0, 2, 1))
    cs = _prefix_table(ft)
    cs_flat = cs.reshape(b * (t + 8), d)
    tois_flat = tois.reshape(b * n, 2).astype(jnp.int32)
    starts = tois_flat[:, 0]
    ends = tois_flat[:, 1]
    rows = _sc_pool(cs_flat, starts, ends, b * n, n, t + 8, d)
    out = rows.reshape(b * n, _NSLOT * d)
    cum = jnp.cumsum(jnp.full((b,), n, dtype=jnp.int32))
    return (out, cum)


# speedup vs baseline: 7.6526x; 1.3667x over previous
import jax
import jax.numpy as jnp
from jax import lax
from jax.experimental import pallas as pl
from jax.experimental.pallas import tpu as pltpu
from jax.experimental.pallas import tpu_sc as plsc

_POOL = 8
_NSLOT = _POOL + 2
_NC, _NS, _LANES = 2, 16, 16
_NW = _NC * _NS


def _cs_body(ft_ref, cs_ref):
    x = ft_ref[0]
    t_len, dblk = x.shape
    ri = lax.broadcasted_iota(jnp.int32, (128, 128), 0)
    ci = lax.broadcasted_iota(jnp.int32, (128, 128), 1)
    ltri = (ri >= ci).astype(jnp.float32)
    carry = jnp.zeros((1, dblk), jnp.float32)
    pieces = [jnp.zeros((1, dblk), jnp.float32)]
    for k in range(t_len // 128):
        ik = jnp.dot(ltri, x[k * 128:(k + 1) * 128],
                     precision=lax.Precision.HIGHEST)
        blk = ik + carry
        pieces.append(blk)
        carry = blk[127:128, :]
    pieces.append(jnp.zeros((7, dblk), jnp.float32))
    cs_ref[0] = jnp.concatenate(pieces, axis=0)


def _prefix_table(ft):
    b, t, d = ft.shape
    dblk = 256
    return pl.pallas_call(
        _cs_body,
        grid=(b, d // dblk),
        in_specs=[pl.BlockSpec((1, t, dblk), lambda i, j: (i, 0, j))],
        out_specs=pl.BlockSpec((1, t + 8, dblk), lambda i, j: (i, 0, j)),
        out_shape=jax.ShapeDtypeStruct((b, t + 8, d), jnp.float32),
    )(ft)


def _sc_pool(cs_flat, starts, ends, n_span, n_per_batch, cs_rows_per_batch, d):
    spw = n_span // _NW
    rpw = spw * _NSLOT
    krows = 16
    nchunk = rpw // krows

    mesh = plsc.VectorSubcoreMesh(core_axis_name="c", subcore_axis_name="s")

    def body(cs_hbm, st_hbm, en_hbm, out_hbm, st_v, en_v, cb_v, sc_v,
             buf0, buf1, gs0, gs1, ws0, ws1):
        wid = lax.axis_index("s") * _NC + lax.axis_index("c")
        span0 = wid * spw
        row0 = wid * rpw
        csbase = (span0 // n_per_batch) * cs_rows_per_batch
        pltpu.sync_copy(st_hbm.at[pl.ds(span0, spw)], st_v)
        pltpu.sync_copy(en_hbm.at[pl.ds(span0, spw)], en_v)

        zero16 = jnp.zeros((_LANES,), jnp.int32)
        onef = jnp.ones((_LANES,), jnp.float32)
        zerof = jnp.zeros((_LANES,), jnp.float32)
        bufs = (buf0, buf1)
        gsems = (gs0, gs1)
        wsems = (ws0, ws1)

        def gbody(g, carry):
            sl = g * _LANES + lax.iota(jnp.int32, _LANES)
            start = st_v[pl.ds(g * _LANES, _LANES)]
            end = en_v[pl.ds(g * _LANES, _LANES)]
            ln = end - start
            m = jnp.maximum(ln - 2, 0)
            lm = m >> 3
            lr = m & 7
            big = ln > _POOL + 2
            inv = 1.0 / jnp.maximum(lm + lr, 1).astype(jnp.float32)
            base = sl
            for s in range(_NSLOT):
                if s == 0:
                    hi, lo, sc = start + 1, start, onef
                elif s == _NSLOT - 1:
                    p = ln > 1
                    hi = jnp.where(p, end, 0)
                    lo = jnp.where(p, end - 1, 0)
                    sc = jnp.where(p, onef, zerof)
                else:
                    exact = (ln >= s + 2) & (ln <= _POOL + 2)
                    hi = jnp.where(big, start + 1 + s * lm + lr,
                                   jnp.where(exact, start + s + 1, 0))
                    lo = jnp.where(big, start + 1 + (s - 1) * lm,
                                   jnp.where(exact, start + s, 0))
                    sc = jnp.where(big, inv, jnp.where(exact, onef, zerof))
                pos = s * spw + base
                ph = ((pos >> 4) << 5) + (pos & 15)
                plsc.store_scatter(cb_v, [ph], hi + csbase)
                plsc.store_scatter(cb_v, [ph + 16], lo + csbase)
                plsc.store_scatter(sc_v, [pos], sc)
            return carry

        lax.fori_loop(0, spw // _LANES, gbody, 0)

        def gather_of(c, par):
            return pltpu.make_async_copy(
                cs_hbm.at[cb_v.at[pl.ds(c * 2 * krows, 2 * krows)]],
                bufs[par], gsems[par])

        def write_of(c, par):
            s = c >> 3
            g = c & 7
            return pltpu.make_async_copy(
                bufs[par].at[pl.ds(0, krows)],
                out_hbm.at[pl.ds(span0 + g * krows, krows), pl.ds(s * d, d)],
                wsems[par])

        gather_of(0, 0).start()

        def cbody(cc, carry):
            for par in range(2):
                c = cc * 2 + par
                nxt = c + 1
                @pl.when(nxt < nchunk)
                def _():
                    @pl.when(c >= 1)
                    def _():
                        write_of(nxt - 2, 1 - par).wait()
                    gather_of(nxt, 1 - par).start()

                gather_of(c, par).wait()

                def rbody(r, carry2):
                    sv = plsc.load_gather(sc_v, [zero16 + (c * krows + r)])
                    for j in range(d // _LANES):
                        h = bufs[par][r, pl.ds(j * _LANES, _LANES)]
                        l = bufs[par][r + krows, pl.ds(j * _LANES, _LANES)]
                        bufs[par][r, pl.ds(j * _LANES, _LANES)] = (h - l) * sv
                    return carry2

                lax.fori_loop(0, krows, rbody, 0)
                write_of(c, par).start()
            return carry

        lax.fori_loop(0, nchunk // 2, cbody, 0)
        write_of(nchunk - 2, 0).wait()
        write_of(nchunk - 1, 1).wait()

    return pl.kernel(
        body,
        mesh=mesh,
        compiler_params=pltpu.CompilerParams(needs_layout_passes=False),
        out_type=jax.ShapeDtypeStruct((n_span, _NSLOT * d), jnp.float32),
        scratch_types=[
            pltpu.VMEM((spw,), jnp.int32),
            pltpu.VMEM((spw,), jnp.int32),
            pltpu.VMEM((nchunk * 2 * krows,), jnp.int32),
            pltpu.VMEM((nchunk * krows,), jnp.float32),
            pltpu.VMEM((2 * krows, d), jnp.float32),
            pltpu.VMEM((2 * krows, d), jnp.float32),
            pltpu.SemaphoreType.DMA,
            pltpu.SemaphoreType.DMA,
            pltpu.SemaphoreType.DMA,
            pltpu.SemaphoreType.DMA,
        ],
    )(cs_flat, starts, ends)


def kernel(features, tois, embeddings):
    b, d, t = features.shape
    n = tois.shape[1]
    ft = jnp.transpose(features, (0, 2, 1))
    cs = _prefix_table(ft)
    cs_flat = cs.reshape(b * (t + 8), d)
    tois_flat = tois.reshape(b * n, 2).astype(jnp.int32)
    starts = tois_flat[:, 0]
    ends = tois_flat[:, 1]
    out = _sc_pool(cs_flat, starts, ends, b * n, n, t + 8, d)
    cum = jnp.cumsum(jnp.full((b,), n, dtype=jnp.int32))
    return (out, cum)


# speedup vs baseline: 9.1782x; 1.1993x over previous
import jax
import jax.numpy as jnp
from jax import lax
from jax.experimental import pallas as pl
from jax.experimental.pallas import tpu as pltpu
from jax.experimental.pallas import tpu_sc as plsc

_POOL = 8
_NSLOT = _POOL + 2
_NC, _NS, _LANES = 2, 16, 16
_NW = _NC * _NS


def _cs_body(ft_ref, cs_ref):
    x = ft_ref[0]
    t_len, dblk = x.shape
    ri = lax.broadcasted_iota(jnp.int32, (128, 128), 0)
    ci = lax.broadcasted_iota(jnp.int32, (128, 128), 1)
    ltri = (ri >= ci).astype(jnp.float32)
    carry = jnp.zeros((1, dblk), jnp.float32)
    pieces = [jnp.zeros((1, dblk), jnp.float32)]
    for k in range(t_len // 128):
        ik = jnp.dot(ltri, x[k * 128:(k + 1) * 128],
                     precision=lax.Precision.HIGHEST)
        blk = ik + carry
        pieces.append(blk)
        carry = blk[127:128, :]
    pieces.append(jnp.zeros((7, dblk), jnp.float32))
    cs_ref[0] = jnp.concatenate(pieces, axis=0)


def _prefix_table(ft):
    b, t, d = ft.shape
    dblk = 256
    return pl.pallas_call(
        _cs_body,
        grid=(b, d // dblk),
        in_specs=[pl.BlockSpec((1, t, dblk), lambda i, j: (i, 0, j))],
        out_specs=pl.BlockSpec((1, t + 8, dblk), lambda i, j: (i, 0, j)),
        out_shape=jax.ShapeDtypeStruct((b, t + 8, d), jnp.float32),
    )(ft)


def _sc_pool(cs_flat, starts, ends, n_span, n_per_batch, cs_rows_per_batch, d):
    spw = n_span // _NW
    rpw = spw * _NSLOT
    krows = 16
    nchunk = rpw // krows

    mesh = plsc.VectorSubcoreMesh(core_axis_name="c", subcore_axis_name="s")

    def body(cs_hbm, st_hbm, en_hbm, out_hbm, st_v, en_v, cb_v, sc_v,
             buf0, buf1, gs0, gs1, ws0, ws1):
        wid = lax.axis_index("s") * _NC + lax.axis_index("c")
        span0 = wid * spw
        row0 = wid * rpw
        csbase = (span0 // n_per_batch) * cs_rows_per_batch
        pltpu.sync_copy(st_hbm.at[pl.ds(span0, spw)], st_v)
        pltpu.sync_copy(en_hbm.at[pl.ds(span0, spw)], en_v)

        zero16 = jnp.zeros((_LANES,), jnp.int32)
        onef = jnp.ones((_LANES,), jnp.float32)
        zerof = jnp.zeros((_LANES,), jnp.float32)
        bufs = (buf0, buf1)
        gsems = (gs0, gs1)
        wsems = (ws0, ws1)

        def gbody(g, carry):
            sl = g * _LANES + lax.iota(jnp.int32, _LANES)
            start = st_v[pl.ds(g * _LANES, _LANES)]
            end = en_v[pl.ds(g * _LANES, _LANES)]
            ln = end - start
            m = jnp.maximum(ln - 2, 0)
            lm = m >> 3
            lr = m & 7
            big = ln > _POOL + 2
            inv = 1.0 / jnp.maximum(lm + lr, 1).astype(jnp.float32)
            base = sl
            for s in range(_NSLOT):
                if s == 0:
                    hi, lo, sc = start + 1, start, onef
                elif s == _NSLOT - 1:
                    p = ln > 1
                    hi = jnp.where(p, end, 0)
                    lo = jnp.where(p, end - 1, 0)
                    sc = jnp.where(p, onef, zerof)
                else:
                    exact = (ln >= s + 2) & (ln <= _POOL + 2)
                    hi = jnp.where(big, start + 1 + s * lm + lr,
                                   jnp.where(exact, start + s + 1, 0))
                    lo = jnp.where(big, start + 1 + (s - 1) * lm,
                                   jnp.where(exact, start + s, 0))
                    sc = jnp.where(big, inv, jnp.where(exact, onef, zerof))
                pos = s * spw + base
                ph = ((pos >> 4) << 5) + (pos & 15)
                plsc.store_scatter(cb_v, [ph], hi + csbase)
                plsc.store_scatter(cb_v, [ph + 16], lo + csbase)
                plsc.store_scatter(sc_v, [pos], sc)
            return carry

        lax.fori_loop(0, spw // _LANES, gbody, 0)

        def gather_of(c, par):
            return pltpu.make_async_copy(
                cs_hbm.at[cb_v.at[pl.ds(c * 2 * krows, 2 * krows)]],
                bufs[par], gsems[par])

        def write_of(c, par):
            s = c >> 3
            g = c & 7
            return pltpu.make_async_copy(
                bufs[par].at[pl.ds(0, krows)],
                out_hbm.at[pl.ds(span0 + g * krows, krows), pl.ds(s * d, d)],
                wsems[par])

        gather_of(0, 0).start()

        def cbody(cc, carry):
            for par in range(2):
                c = cc * 2 + par
                nxt = c + 1
                @pl.when(nxt < nchunk)
                def _():
                    @pl.when(c >= 1)
                    def _():
                        write_of(nxt - 2, 1 - par).wait()
                    gather_of(nxt, 1 - par).start()

                gather_of(c, par).wait()

                def rbody(r, carry2):
                    sv = plsc.load_gather(sc_v, [zero16 + (c * krows + r)])

                    def jbody(j, carry3):
                        for u in range(16):
                            off = j * 16 * _LANES + u * _LANES
                            h = bufs[par][r, pl.ds(off, _LANES)]
                            l = bufs[par][r + krows, pl.ds(off, _LANES)]
                            bufs[par][r, pl.ds(off, _LANES)] = (h - l) * sv
                        return carry3

                    lax.fori_loop(0, d // _LANES // 16, jbody, 0)
                    return carry2

                lax.fori_loop(0, krows, rbody, 0)
                write_of(c, par).start()
            return carry

        lax.fori_loop(0, nchunk // 2, cbody, 0)
        write_of(nchunk - 2, 0).wait()
        write_of(nchunk - 1, 1).wait()

    return pl.kernel(
        body,
        mesh=mesh,
        compiler_params=pltpu.CompilerParams(needs_layout_passes=False),
        out_type=jax.ShapeDtypeStruct((n_span, _NSLOT * d), jnp.float32),
        scratch_types=[
            pltpu.VMEM((spw,), jnp.int32),
            pltpu.VMEM((spw,), jnp.int32),
            pltpu.VMEM((nchunk * 2 * krows,), jnp.int32),
            pltpu.VMEM((nchunk * krows,), jnp.float32),
            pltpu.VMEM((2 * krows, d), jnp.float32),
            pltpu.VMEM((2 * krows, d), jnp.float32),
            pltpu.SemaphoreType.DMA,
            pltpu.SemaphoreType.DMA,
            pltpu.SemaphoreType.DMA,
            pltpu.SemaphoreType.DMA,
        ],
    )(cs_flat, starts, ends)


def kernel(features, tois, embeddings):
    b, d, t = features.shape
    n = tois.shape[1]
    ft = jnp.transpose(features, (0, 2, 1))
    cs = _prefix_table(ft)
    cs_flat = cs.reshape(b * (t + 8), d)
    tois_flat = tois.reshape(b * n, 2).astype(jnp.int32)
    starts = tois_flat[:, 0]
    ends = tois_flat[:, 1]
    out = _sc_pool(cs_flat, starts, ends, b * n, n, t + 8, d)
    cum = jnp.cumsum(jnp.full((b,), n, dtype=jnp.int32))
    return (out, cum)
